# Initial kernel scaffold; baseline (speedup 1.0000x reference)
import jax
import jax.numpy as jnp
from jax.experimental import pallas as pl


def kernel(question, emb, W1_w, W1_b):
    raise NotImplementedError("write your pallas kernel here")


# speedup vs baseline: 2.9128x; 2.9128x over previous
import functools

import jax
import jax.numpy as jnp
from jax import lax
from jax.experimental import pallas as pl
from jax.experimental.pallas import tpu as pltpu
from jax.experimental.pallas import tpu_sc as plsc

NC, NS, LANES = 2, 16, 16
NW = NC * NS
CB = 8


def _sc_stage(B, H, D):
    BPW = B // NW
    NBLK = BPW // CB
    IPB = CB * H
    splits = []
    t = 0
    while t < IPB:
        ln = min(128, IPB - t)
        splits.append((t, ln))
        t += ln
    splits = tuple(splits)

    def body(qflat, emb, sums, sumsq,
             idx_v, rows0, rows1, osum, osq, sem0, sem1):
        wid = lax.axis_index("s") * NC + lax.axis_index("c")
        base = wid * BPW
        ibase = pl.multiple_of(base * H, 8)
        pltpu.sync_copy(qflat.at[pl.ds(ibase, BPW * H)], idx_v)

        bufs = (rows0, rows1)
        sems = (sem0, sem1)

        def streams(j, buf, sem):
            off = j * IPB
            out = []
            for t, ln in splits:
                out.append((
                    emb.at[idx_v.at[pl.ds(pl.multiple_of(off + t, 8), ln)]],
                    buf.at[pl.ds(t, ln)], sem))
            return out

        def fire(j, buf, sem):
            for src, dst, s in streams(j, buf, sem):
                pltpu.async_copy(src, dst, s)

        def drain(j, buf, sem):
            for src, dst, s in streams(j, buf, sem):
                pltpu.make_async_copy(src, dst, s).wait()

        zero = jnp.zeros((LANES,), jnp.float32)

        def compute(j, buf):
            ob = j * CB
            for i in range(CB):
                rbase = i * H

                def rbody(l, c):
                    s0, s1, q0, q1 = c
                    r = rbase + l
                    v0 = buf[r, pl.ds(0, LANES)]
                    v1 = buf[r, pl.ds(LANES, LANES)]
                    return (s0 + v0, s1 + v1, q0 + v0 * v0, q1 + v1 * v1)

                s0, s1, q0, q1 = lax.fori_loop(
                    0, H, rbody, (zero, zero, zero, zero), unroll=5)
                row = ob + i
                osum[row, pl.ds(0, LANES)] = s0
                osum[row, pl.ds(LANES, LANES)] = s1
                osq[row, pl.ds(0, LANES)] = q0
                osq[row, pl.ds(LANES, LANES)] = q1

        fire(0, bufs[0], sems[0])
        fire(1, bufs[1], sems[1])

        def pair(p, carry):
            for b in range(2):
                j = p * 2 + b
                drain(j, bufs[b], sems[b])
                compute(j, bufs[b])
                nj = j + 2

                @pl.when(nj < NBLK)
                def _():
                    fire(nj, bufs[b], sems[b])
            return carry

        lax.fori_loop(0, NBLK // 2, pair, 0)

        obase = pl.multiple_of(base, 8)
        pltpu.sync_copy(osum, sums.at[pl.ds(obase, BPW)])
        pltpu.sync_copy(osq, sumsq.at[pl.ds(obase, BPW)])

    return pl.kernel(
        body,
        out_type=[jax.ShapeDtypeStruct((B, D), jnp.float32),
                  jax.ShapeDtypeStruct((B, D), jnp.float32)],
        mesh=plsc.VectorSubcoreMesh(core_axis_name="c", subcore_axis_name="s"),
        compiler_params=pltpu.CompilerParams(use_tc_tiling_on_sc=False),
        scratch_types=[
            pltpu.VMEM((BPW * H,), jnp.int32),
            pltpu.VMEM((IPB, D), jnp.float32),
            pltpu.VMEM((IPB, D), jnp.float32),
            pltpu.VMEM((BPW, D), jnp.float32),
            pltpu.VMEM((BPW, D), jnp.float32),
            pltpu.SemaphoreType.DMA,
            pltpu.SemaphoreType.DMA,
        ],
    )


def _tc_body(H, s_ref, q_ref, w_ref, b_ref, o_ref):
    s = s_ref[...]
    q = q_ref[...]
    denom = jnp.maximum(jnp.sqrt(q) * jnp.sqrt(jnp.float32(H)), 1e-8)
    h2 = jnp.maximum(s / denom, 0.0)
    w = w_ref[...]
    logit = jnp.sum(h2 * w, axis=1, keepdims=True) + b_ref[0, 0]
    o_ref[...] = 1.0 / (1.0 + jnp.exp(-logit))


def kernel(question, emb, W1_w, W1_b):
    B, H = question.shape
    V, D = emb.shape
    qflat = question.reshape(-1).astype(jnp.int32)
    sums, sumsq = _sc_stage(B, H, D)(qflat, emb)
    out = pl.pallas_call(
        functools.partial(_tc_body, H),
        out_shape=jax.ShapeDtypeStruct((B, 1), jnp.float32),
    )(sums, sumsq, W1_w, W1_b.reshape(1, 1))
    return out
